# baseline (speedup 1.0000x reference)
import functools

import jax
import jax.numpy as jnp
from jax import lax
from jax.experimental import pallas as pl
from jax.experimental.pallas import tpu as pltpu
from jax.experimental.pallas import tpu_sc as plsc

B = 16384
D = 32
NC = 2
NS = 16
NW = NC * NS
BPW = B // NW
CH = 128
NCH = BPW // CH
NBLK = 1000000 * D // 128

_mesh = plsc.VectorSubcoreMesh(core_axis_name="c", subcore_axis_name="s")


@functools.partial(
    pl.kernel,
    mesh=_mesh,
    out_type=[
        jax.ShapeDtypeStruct((B,), jnp.float32),
        jax.ShapeDtypeStruct((B,), jnp.float32),
    ],
    scratch_types=[
        pltpu.VMEM((BPW,), jnp.int32),
        pltpu.VMEM((BPW,), jnp.int32),
        pltpu.VMEM((BPW,), jnp.int32),
        pltpu.VMEM((BPW,), jnp.int32),
        pltpu.VMEM((BPW,), jnp.int32),
        pltpu.VMEM((BPW,), jnp.int32),
        pltpu.VMEM((2, CH, 128), jnp.float32),
        pltpu.VMEM((2, CH, 128), jnp.float32),
        pltpu.VMEM((2, CH, 128), jnp.float32),
        pltpu.VMEM((BPW,), jnp.float32),
        pltpu.VMEM((BPW,), jnp.float32),
        pltpu.SemaphoreType.DMA,
        pltpu.SemaphoreType.DMA,
        pltpu.SemaphoreType.DMA,
        pltpu.SemaphoreType.DMA,
        pltpu.SemaphoreType.DMA,
        pltpu.SemaphoreType.DMA,
    ],
)
def _bpr_sc(ublk_hbm, iblk_hbm, jblk_hbm, uoff_hbm, ioff_hbm, joff_hbm,
            uw_hbm, iw_hbm, out_i_hbm, out_j_hbm,
            uidx, iidx, jidx, uoff, ioff, joff,
            ubuf, ibuf, jbuf, oi, oj,
            su0, si0, sj0, su1, si1, sj1):
    wid = lax.axis_index("s") * NC + lax.axis_index("c")
    base = wid * BPW

    pltpu.sync_copy(ublk_hbm.at[pl.ds(base, BPW)], uidx)
    pltpu.sync_copy(iblk_hbm.at[pl.ds(base, BPW)], iidx)
    pltpu.sync_copy(jblk_hbm.at[pl.ds(base, BPW)], jidx)
    pltpu.sync_copy(uoff_hbm.at[pl.ds(base, BPW)], uoff)
    pltpu.sync_copy(ioff_hbm.at[pl.ds(base, BPW)], ioff)
    pltpu.sync_copy(joff_hbm.at[pl.ds(base, BPW)], joff)

    sems = [(su0, si0, sj0), (su1, si1, sj1)]

    def fire(c):
        slot = c % 2
        su, si, sj = sems[slot]
        sl = pl.ds(c * CH, CH)
        return (
            pltpu.async_copy(uw_hbm.at[uidx.at[sl]], ubuf.at[slot], su),
            pltpu.async_copy(iw_hbm.at[iidx.at[sl]], ibuf.at[slot], si),
            pltpu.async_copy(iw_hbm.at[jidx.at[sl]], jbuf.at[slot], sj),
        )

    lanes = lax.iota(jnp.int32, 16)
    perms = [lanes ^ (1 << k) for k in range(4)]

    def hsum(v):
        for p in perms:
            v = v + v.at[p].get(mode="promise_in_bounds")
        return v

    pending = fire(0)
    for c in range(NCH):
        nxt = fire(c + 1) if c + 1 < NCH else None
        for cp in pending:
            cp.wait()
        slot = c % 2

        def grp_body(g, carry, c=c, slot=slot):
            acc_i = jnp.zeros((16,), jnp.float32)
            acc_j = jnp.zeros((16,), jnp.float32)
            gbase = c * CH + g * 16
            qvu = uoff[pl.ds(gbase, 16)]
            qvi = ioff[pl.ds(gbase, 16)]
            qvj = joff[pl.ds(gbase, 16)]
            for k in range(16):
                r = g * 16 + k
                qu = qvu[k]
                qi = qvi[k]
                qj = qvj[k]
                u0 = ubuf[slot, r, pl.ds(qu, 16)]
                u1 = ubuf[slot, r, pl.ds(qu + 16, 16)]
                i0 = ibuf[slot, r, pl.ds(qi, 16)]
                i1 = ibuf[slot, r, pl.ds(qi + 16, 16)]
                j0 = jbuf[slot, r, pl.ds(qj, 16)]
                j1 = jbuf[slot, r, pl.ds(qj + 16, 16)]
                si = hsum(u0 * i0 + u1 * i1)
                sj = hsum(u0 * j0 + u1 * j1)
                m = lanes == k
                acc_i = jnp.where(m, si, acc_i)
                acc_j = jnp.where(m, sj, acc_j)
            oi[pl.ds(c * CH + g * 16, 16)] = acc_i
            oj[pl.ds(c * CH + g * 16, 16)] = acc_j
            return carry

        lax.fori_loop(0, CH // 16, grp_body, 0)
        pending = nxt

    pltpu.sync_copy(oi, out_i_hbm.at[pl.ds(base, BPW)])
    pltpu.sync_copy(oj, out_j_hbm.at[pl.ds(base, BPW)])


def kernel(user, item_i, item_j, embed_user_weight, embed_item_weight):
    user = user.astype(jnp.int32)
    item_i = item_i.astype(jnp.int32)
    item_j = item_j.astype(jnp.int32)
    ublk = user >> 2
    iblk = item_i >> 2
    jblk = item_j >> 2
    uoff = (user & 3) << 5
    ioff = (item_i & 3) << 5
    joff = (item_j & 3) << 5
    uw = embed_user_weight.reshape(NBLK, 128)
    iw = embed_item_weight.reshape(NBLK, 128)
    pi, pj = _bpr_sc(ublk, iblk, jblk, uoff, ioff, joff, uw, iw)
    return pi, pj


# speedup vs baseline: 1.0121x; 1.0121x over previous
import functools

import jax
import jax.numpy as jnp
from jax import lax
from jax.experimental import pallas as pl
from jax.experimental.pallas import tpu as pltpu
from jax.experimental.pallas import tpu_sc as plsc

B = 16384
D = 32
NC = 2
NS = 16
NW = NC * NS
BPW = B // NW
ICH = 128
NCH = BPW // ICH

_mesh = plsc.VectorSubcoreMesh(core_axis_name="c", subcore_axis_name="s")


@functools.partial(
    pl.kernel,
    mesh=_mesh,
    compiler_params=pltpu.CompilerParams(use_tc_tiling_on_sc=False),
    out_type=[
        jax.ShapeDtypeStruct((NW, BPW), jnp.float32),
        jax.ShapeDtypeStruct((NW, BPW), jnp.float32),
    ],
    scratch_types=[
        pltpu.VMEM((NCH, ICH), jnp.int32),
        pltpu.VMEM((NCH, ICH), jnp.int32),
        pltpu.VMEM((NCH, ICH), jnp.int32),
        pltpu.VMEM((BPW, D), jnp.float32),
        pltpu.VMEM((BPW, D), jnp.float32),
        pltpu.VMEM((BPW, D), jnp.float32),
        pltpu.VMEM((BPW,), jnp.float32),
        pltpu.VMEM((BPW,), jnp.float32),
        pltpu.SemaphoreType.DMA,
        pltpu.SemaphoreType.DMA,
        pltpu.SemaphoreType.DMA,
    ],
)
def _bpr_sc(user_hbm, item_i_hbm, item_j_hbm, uw_hbm, iw_hbm,
            out_i_hbm, out_j_hbm,
            uidx, iidx, jidx, urows, irows, jrows, oi, oj,
            su, si, sj):
    wid = lax.axis_index("s") * NC + lax.axis_index("c")

    pltpu.sync_copy(user_hbm.at[wid], uidx)
    pltpu.sync_copy(item_i_hbm.at[wid], iidx)
    pltpu.sync_copy(item_j_hbm.at[wid], jidx)

    copies = []
    for k in range(NCH):
        sl = pl.ds(k * ICH, ICH)
        copies.append(pltpu.async_copy(uw_hbm.at[uidx.at[k]], urows.at[sl], su))
        copies.append(pltpu.async_copy(iw_hbm.at[iidx.at[k]], irows.at[sl], si))
        copies.append(pltpu.async_copy(iw_hbm.at[jidx.at[k]], jrows.at[sl], sj))
    for c in copies:
        c.wait()

    lanes = lax.iota(jnp.int32, 16)
    perms = [lanes ^ (1 << k) for k in range(4)]

    def hsum(v):
        for p in perms:
            v = v + v.at[p].get(mode="promise_in_bounds")
        return v

    def grp_body(g, carry):
        acc_i = jnp.zeros((16,), jnp.float32)
        acc_j = jnp.zeros((16,), jnp.float32)
        for k in range(16):
            r = g * 16 + k
            u0 = urows[r, pl.ds(0, 16)]
            u1 = urows[r, pl.ds(16, 16)]
            i0 = irows[r, pl.ds(0, 16)]
            i1 = irows[r, pl.ds(16, 16)]
            j0 = jrows[r, pl.ds(0, 16)]
            j1 = jrows[r, pl.ds(16, 16)]
            si_ = hsum(u0 * i0 + u1 * i1)
            sj_ = hsum(u0 * j0 + u1 * j1)
            m = lanes == k
            acc_i = jnp.where(m, si_, acc_i)
            acc_j = jnp.where(m, sj_, acc_j)
        oi[pl.ds(g * 16, 16)] = acc_i
        oj[pl.ds(g * 16, 16)] = acc_j
        return carry

    lax.fori_loop(0, BPW // 16, grp_body, 0)

    pltpu.sync_copy(oi, out_i_hbm.at[wid])
    pltpu.sync_copy(oj, out_j_hbm.at[wid])


def kernel(user, item_i, item_j, embed_user_weight, embed_item_weight):
    u = user.astype(jnp.int32).reshape(NW, NCH, ICH)
    ii = item_i.astype(jnp.int32).reshape(NW, NCH, ICH)
    ij = item_j.astype(jnp.int32).reshape(NW, NCH, ICH)
    pi, pj = _bpr_sc(u, ii, ij, embed_user_weight, embed_item_weight)
    return pi.reshape(B), pj.reshape(B)
